# Initial kernel scaffold; baseline (speedup 1.0000x reference)
import jax
import jax.numpy as jnp
from jax.experimental import pallas as pl


def kernel(inputs, params):
    raise NotImplementedError("write your pallas kernel here")


# speedup vs baseline: 2.1740x; 2.1740x over previous
import functools

import jax
import jax.numpy as jnp
from jax import lax
from jax.experimental import pallas as pl
from jax.experimental.pallas import tpu as pltpu

_CH = [128, 256, 512, 1024]
_BLK = [2, 2, 4, 2]
_RR = [4, 2, 1, 1]
_K = 9
_MAXD = 5

_HI = lax.Precision.HIGHEST
_DEF = lax.Precision.DEFAULT


def _row_tile(n):
    for r in (512, 256, 192, 144, 128, 96, 64, 48, 32, 16, 8):
        if n % r == 0:
            return r
    return n


def _mm_body(nx, precision, *refs):
    xs = refs[:nx]
    ws = refs[nx:2 * nx]
    y_ref = refs[2 * nx]
    acc = None
    for x, w in zip(xs, ws):
        t = jnp.dot(x[...], w[...], preferred_element_type=jnp.float32,
                    precision=precision)
        acc = t if acc is None else acc + t
    y_ref[...] = acc


def _mm(xs, ws, precision=_DEF):
    n = xs[0].shape[0]
    co = ws[0].shape[1]
    r = _row_tile(n)
    tc = min(512, co)
    ni, nj = n // r, co // tc
    in_specs = []
    for x in xs:
        k = x.shape[1]
        in_specs.append(pl.BlockSpec((r, k), lambda j, i: (i, 0)))
    for w in ws:
        k = w.shape[0]
        in_specs.append(pl.BlockSpec((k, tc), lambda j, i: (0, j)))
    return pl.pallas_call(
        functools.partial(_mm_body, len(xs), precision),
        grid=(nj, ni),
        in_specs=in_specs,
        out_specs=pl.BlockSpec((r, tc), lambda j, i: (i, j)),
        out_shape=jax.ShapeDtypeStruct((n, co), jnp.float32),
    )(*xs, *ws)


def _bn_aux(y, hh, ww, g, be):
    t = jnp.transpose(y, (1, 0)).reshape(1, -1, hh, ww)
    t = lax.optimization_barrier(t)
    m = jnp.mean(t, axis=(0, 2, 3))
    v = jnp.var(t, axis=(0, 2, 3))
    c = m.shape[0]
    return jnp.concatenate(
        [m[None], v[None], g[None], be[None], jnp.zeros((4, c), jnp.float32)], 0)


def _bnact_body(act, has_res, *refs):
    if has_res:
        y_ref, ss_ref, r_ref, o_ref = refs
    else:
        y_ref, ss_ref, o_ref = refs
    m = ss_ref[0:1, :]
    var = ss_ref[1:2, :]
    g = ss_ref[2:3, :]
    be = ss_ref[3:4, :]
    v = (y_ref[...] - m) / jnp.sqrt(var + 1e-5) * g + be
    if act == 'gelu':
        v = jax.nn.gelu(v)
    elif act == 'leaky':
        v = jnp.where(v >= 0, v, 0.01 * v)
    if has_res:
        v = v + r_ref[...]
    o_ref[...] = v


def _bnact(y, ss, act=None, res=None):
    n, c = y.shape
    r = _row_tile(n)
    ins = [y, ss] + ([res] if res is not None else [])
    in_specs = [pl.BlockSpec((r, c), lambda i: (i, 0)),
                pl.BlockSpec((8, c), lambda i: (0, 0))]
    if res is not None:
        in_specs.append(pl.BlockSpec((r, c), lambda i: (i, 0)))
    return pl.pallas_call(
        functools.partial(_bnact_body, act, res is not None),
        grid=(n // r,),
        in_specs=in_specs,
        out_specs=pl.BlockSpec((r, c), lambda i: (i, 0)),
        out_shape=jax.ShapeDtypeStruct((n, c), jnp.float32),
    )(*ins)


def _graph_body(kd, d, m_sz, *refs):
    x_ref, yf_ref, rel_ref, g_ref = refs
    x = x_ref[...]
    yf = yf_ref[...]
    r, c = x.shape
    yn = yf / (jnp.sqrt(jnp.sum(yf * yf, axis=1, keepdims=True)) + 1e-12)
    xn = x / (jnp.sqrt(jnp.sum(x * x, axis=1, keepdims=True)) + 1e-12)
    a = jnp.sum(xn * xn, axis=1, keepdims=True)
    b8 = lax.dot_general(jnp.ones((8, c), jnp.float32), yn * yn,
                         (((1,), (1,)), ((), ())), precision=_HI)
    s = lax.dot_general(xn, yn, (((1,), (1,)), ((), ())), precision=_DEF)
    dist = -2.0 * s + a + b8[0:1, :]
    key = -(dist + rel_ref[...])
    cols = lax.broadcasted_iota(jnp.int32, (r, m_sz), 1)
    g = None
    for j in range(kd):
        rm = jnp.max(key, axis=1, keepdims=True)
        eq = key == rm
        mc = jnp.where(eq, cols, m_sz)
        first = jnp.min(mc, axis=1, keepdims=True)
        oh = cols == first
        if j % d == 0:
            ohf = oh.astype(jnp.float32)
            gj = jnp.dot(ohf, yf, preferred_element_type=jnp.float32,
                         precision=_HI)
            g = gj if g is None else jnp.maximum(g, gj)
        if j < kd - 1:
            key = jnp.where(oh, -jnp.inf, key)
    g_ref[...] = g - x


def _graph_core(xf, yf, rel, d):
    n, c = xf.shape
    m = yf.shape[0]
    r = min(256, _row_tile(n))
    kd = _K * d
    return pl.pallas_call(
        functools.partial(_graph_body, kd, d, m),
        grid=(n // r,),
        in_specs=[
            pl.BlockSpec((r, c), lambda i: (i, 0)),
            pl.BlockSpec((m, c), lambda i: (0, 0)),
            pl.BlockSpec((r, m), lambda i: (i, 0)),
        ],
        out_specs=pl.BlockSpec((r, c), lambda i: (i, 0)),
        out_shape=jax.ShapeDtypeStruct((n, c), jnp.float32),
    )(xf, yf, rel)


def _im2col(x, stride):
    h, w, c = x.shape
    xp = jnp.pad(x, ((1, 1), (1, 1), (0, 0)))
    ho = (h - 1) // stride + 1
    wo = (w - 1) // stride + 1
    cols = []
    for dy in range(3):
        for dx in range(3):
            sl = lax.slice(xp, (dy, dx, 0),
                           (dy + (ho - 1) * stride + 1,
                            dx + (wo - 1) * stride + 1, c),
                           (stride, stride, 1))
            cols.append(sl)
    return jnp.concatenate(cols, axis=-1).reshape(ho * wo, 9 * c), ho, wo


def _conv3x3_bn(x, w_oihw, g, be, stride, act=None, res=None):
    cm, ho, wo = _im2col(x, stride)
    co = w_oihw.shape[0]
    wmat = jnp.transpose(w_oihw, (2, 3, 1, 0)).reshape(cm.shape[1], co)
    if cm.shape[1] % 8:
        padk = 8 - cm.shape[1] % 8
        cm = jnp.pad(cm, ((0, 0), (0, padk)))
        wmat = jnp.pad(wmat, ((0, padk), (0, 0)))
    y = _mm([cm], [wmat])
    a = _bnact(y, _bn_aux(y, ho, wo, g, be), act, res)
    return a, ho, wo


def _pool_matrix(h, r):
    n = h * h
    hp = h // r
    i = jnp.arange(n)
    row_of_n = (i // h) // r * hp + (i % h) // r
    p = (row_of_n[None, :] == jnp.arange(hp * hp)[:, None]).astype(jnp.float32)
    return p / (r * r)


def _grapher(a, bp, h, r, d):
    n, c = a.shape
    w1 = bp['fc1_w'][:, :, 0, 0].T
    y1 = _mm([a], [w1])
    xf = _bnact(y1, _bn_aux(y1, h, h, bp['fc1_g'], bp['fc1_be']))
    if r > 1:
        p = _pool_matrix(h, r)
        yf = _mm([p], [xf], precision=_HI)
    else:
        yf = xf
    mj = _graph_core(xf, yf, bp['rel_pos'], d)
    z = jnp.stack([xf, mj], axis=2).reshape(n, 2 * c)
    wmr = bp['mr_w'][:, :, 0, 0].T
    y2 = _mm([z], [wmr])
    zz = _bnact(y2, _bn_aux(y2, h, h, bp['mr_g'], bp['mr_be']), 'gelu')
    w2 = bp['fc2_w'][:, :, 0, 0].T
    y3 = _mm([zz], [w2])
    return _bnact(y3, _bn_aux(y3, h, h, bp['fc2_g'], bp['fc2_be']), None, res=a)


def _ffn(a, bp, h):
    y1 = _mm([a], [bp['f1_w'][:, :, 0, 0].T])
    t = _bnact(y1, _bn_aux(y1, h, h, bp['f1_g'], bp['f1_be']), 'gelu')
    y2 = _mm([t], [bp['f2_w'][:, :, 0, 0].T])
    return _bnact(y2, _bn_aux(y2, h, h, bp['f2_g'], bp['f2_be']), None, res=a)


def kernel(inputs, params):
    xh = jnp.transpose(inputs[0], (1, 2, 0))
    st = params['stem']
    a1, h, w = _conv3x3_bn(xh, st['w1'], st['g1'], st['be1'], 2, 'gelu')
    a1 = a1.reshape(h, w, -1)
    a2, h, w = _conv3x3_bn(a1, st['w2'], st['g2'], st['be2'], 2, 'gelu')
    a2 = a2.reshape(h, w, -1)
    pos = jnp.transpose(params['pos_embed'][0], (1, 2, 0)).reshape(h * w, -1)
    a3, h, w = _conv3x3_bn(a2, st['w3'], st['g3'], st['be3'], 1, None, res=pos)
    x = a3

    outs = []
    idx = 0
    for s in range(4):
        if s > 0:
            cprev = _CH[s - 1]
            pm = params['pm'][s - 1]
            xi = x.reshape(h, h, cprev)
            am, h, w = _conv3x3_bn(xi, pm['w'], pm['g'], pm['be'], 2, None)
            dn = params['down'][s - 1]
            y = _mm([am], [dn['w'][:, :, 0, 0].T])
            x = _bnact(y, _bn_aux(y, h, h, dn['g'], dn['be']), 'leaky')
        for bp in params['blocks'][s]:
            d = min(idx // 4 + 1, _MAXD)
            x = _grapher(x, bp, h, _RR[s], d)
            x = _ffn(x, bp, h)
            idx += 1
        outs.append(jnp.transpose(x.reshape(h, h, _CH[s]), (2, 0, 1))[None])
    return (outs[3], outs[0], outs[1], outs[2], outs[3])
